# Initial kernel scaffold; baseline (speedup 1.0000x reference)
import functools

import jax
import jax.numpy as jnp
from jax import lax
from jax.experimental import pallas as pl
from jax.experimental.pallas import tpu as pltpu
from jax.experimental.pallas import tpu_sc as plsc

_N = 10000
_E = 320000
_D = 128
_H = 128
_AW = 144

_NC = 2
_NS = 16
_NW = _NC * _NS
_EPW = _E // _NW
_C = 80
_CHUNKS = _EPW // _C
_RPT = 1000
_NWRITERS = _N // _RPT


def _sc_segment_sum(x_aug, src, dst, zeros_blk):
    mesh = plsc.VectorSubcoreMesh(core_axis_name="c", subcore_axis_name="s")

    @functools.partial(
        pl.kernel,
        out_type=jax.ShapeDtypeStruct((_NC, _N, _AW), jnp.float32),
        mesh=mesh,
        scratch_types=[
            pltpu.VMEM((_C,), jnp.int32),
            pltpu.VMEM((_C,), jnp.int32),
            pltpu.VMEM((_C, _AW), jnp.float32),
            pltpu.VMEM_SHARED((_N, _AW), jnp.float32),
            pltpu.SemaphoreType.DMA,
        ],
    )
    def seg_kernel(xa_hbm, src_hbm, dst_hbm, z_hbm, out_hbm,
                   src_v, dst_v, rows_v, acc_sh, sem):
        cid = lax.axis_index("c")
        sid = lax.axis_index("s")
        wid = cid * _NS + sid

        @pl.when(sid < _NWRITERS)
        def _():
            pltpu.sync_copy(z_hbm, acc_sh.at[pl.ds(sid * _RPT, _RPT)])

        plsc.subcore_barrier()

        base = wid * _EPW

        def body(i, carry):
            off = base + i * _C
            pltpu.sync_copy(src_hbm.at[pl.ds(off, _C)], src_v)
            pltpu.sync_copy(dst_hbm.at[pl.ds(off, _C)], dst_v)
            pltpu.async_copy(xa_hbm.at[src_v], rows_v, sem).wait()
            pltpu.sync_copy(rows_v, acc_sh.at[dst_v], add=True)
            return carry

        lax.fori_loop(0, _CHUNKS, body, 0)

        plsc.subcore_barrier()

        @pl.when(sid < _NWRITERS)
        def _():
            pltpu.sync_copy(acc_sh.at[pl.ds(sid * _RPT, _RPT)],
                            out_hbm.at[cid, pl.ds(sid * _RPT, _RPT)])

    return seg_kernel(x_aug, src, dst, zeros_blk)


_BN = 1000


def _tc_gates(acc0, acc1, x, Wls, Wrs, bs, Wpi, Wpc, Wpo, bpi, bpc, bpo):
    def body(a0, a1, xr, wls, wrs, bsr, wpi, wpc, wpo, bpir, bpcr, bpor,
             h_ref, c_ref):
        ssum = a0[:, :_D] + a1[:, :_D]
        cnt = a0[:, _D:_D + 1] + a1[:, _D:_D + 1]
        mean = ssum / jnp.maximum(cnt, 1.0)
        u = (jnp.dot(mean, wls[:, :], preferred_element_type=jnp.float32)
             + jnp.dot(xr[:, :], wrs[:, :], preferred_element_type=jnp.float32)
             + bsr[:, :])
        zi = jnp.dot(u[:, 0:_H], wpi[:, :],
                     preferred_element_type=jnp.float32) + bpir[:, :]
        zc = jnp.dot(u[:, _H:2 * _H], wpc[:, :],
                     preferred_element_type=jnp.float32) + bpcr[:, :]
        zo = jnp.dot(u[:, 2 * _H:3 * _H], wpo[:, :],
                     preferred_element_type=jnp.float32) + bpor[:, :]
        ig = jax.nn.sigmoid(zi)
        tg = jnp.tanh(zc)
        og = jax.nn.sigmoid(zo)
        cn = ig * tg
        c_ref[:, :] = cn
        h_ref[:, :] = og * jnp.tanh(cn)

    row_block = lambda w: pl.BlockSpec((_BN, w), lambda i: (i, 0))
    full = lambda a: pl.BlockSpec(a.shape, lambda i: (0,) * a.ndim)

    return pl.pallas_call(
        body,
        grid=(_N // _BN,),
        in_specs=[
            row_block(_AW), row_block(_AW), row_block(_D),
            full(Wls), full(Wrs), full(bs),
            full(Wpi), full(Wpc), full(Wpo),
            full(bpi), full(bpc), full(bpo),
        ],
        out_specs=[row_block(_H), row_block(_H)],
        out_shape=[
            jax.ShapeDtypeStruct((_N, _H), jnp.float32),
            jax.ShapeDtypeStruct((_N, _H), jnp.float32),
        ],
    )(acc0, acc1, x, Wls, Wrs, bs, Wpi, Wpc, Wpo, bpi, bpc, bpo)


def kernel(x, edge_index,
           Wl_i, Wr_i, b_i, Wp_i, bp_i,
           Wl_f, Wr_f, b_f, Wp_f, bp_f,
           Wl_c, Wr_c, b_c, Wp_c, bp_c,
           Wl_o, Wr_o, b_o, Wp_o, bp_o):
    src = edge_index[0].astype(jnp.int32)
    dst = edge_index[1].astype(jnp.int32)

    x_aug = jnp.concatenate(
        [x, jnp.ones((_N, 1), x.dtype), jnp.zeros((_N, _AW - _D - 1), x.dtype)],
        axis=1)
    zeros_blk = jnp.zeros((_RPT, _AW), jnp.float32)

    acc = _sc_segment_sum(x_aug, src, dst, zeros_blk)

    Wls = jnp.concatenate([Wl_i, Wl_c, Wl_o], axis=1)
    Wrs = jnp.concatenate([Wr_i, Wr_c, Wr_o], axis=1)
    bs = jnp.concatenate([b_i, b_c, b_o])[None, :]

    h_new, c_new = _tc_gates(acc[0], acc[1], x, Wls, Wrs, bs,
                             Wp_i, Wp_c, Wp_o,
                             bp_i[None, :], bp_c[None, :], bp_o[None, :])
    return (h_new, c_new)


# speedup vs baseline: 5.6237x; 5.6237x over previous
import functools

import jax
import jax.numpy as jnp
from jax import lax
from jax.experimental import pallas as pl
from jax.experimental.pallas import tpu as pltpu
from jax.experimental.pallas import tpu_sc as plsc

_N = 10000
_E = 320000
_D = 128
_H = 128

_NC = 2
_NS = 16
_NW = _NC * _NS
_EPW = _E // _NW
_C = 80
_CHUNKS = _EPW // _C
_RPT = 1000
_NWRITERS = _N // _RPT
_L = 16
_NVEC = _N // _L


def _sc_segment_sum(x, src, dst, zeros_blk):
    mesh = plsc.VectorSubcoreMesh(core_axis_name="c", subcore_axis_name="s")

    @functools.partial(
        pl.kernel,
        out_type=[
            jax.ShapeDtypeStruct((_NC, _N, _D), jnp.float32),
            jax.ShapeDtypeStruct((_NC, _N), jnp.float32),
        ],
        mesh=mesh,
        compiler_params=pltpu.CompilerParams(needs_layout_passes=False),
        scratch_types=[
            pltpu.VMEM((_C,), jnp.int32),
            pltpu.VMEM((_C,), jnp.int32),
            pltpu.VMEM((_C, _D), jnp.float32),
            pltpu.VMEM((_N,), jnp.float32),
            pltpu.VMEM((_N,), jnp.float32),
            pltpu.VMEM_SHARED((_N, _D), jnp.float32),
            pltpu.VMEM_SHARED((_NS, _N), jnp.float32),
            pltpu.SemaphoreType.DMA,
        ],
    )
    def seg_kernel(x_hbm, src_hbm, dst_hbm, z_hbm, out_rows, out_cnt,
                   src_v, dst_v, rows_v, cnt_v, tmp_v, acc_sh, cnt_sh, sem):
        cid = lax.axis_index("c")
        sid = lax.axis_index("s")
        wid = cid * _NS + sid

        @pl.when(sid < _NWRITERS)
        def _():
            pltpu.sync_copy(z_hbm, acc_sh.at[pl.ds(sid * _RPT, _RPT)])

        zeros16 = jnp.zeros((_L,), jnp.float32)

        def zbody(k, carry):
            cnt_v[pl.ds(k * _L, _L)] = zeros16
            return carry

        lax.fori_loop(0, _NVEC, zbody, 0)

        plsc.subcore_barrier()

        base = wid * _EPW
        ones16 = jnp.ones((_L,), jnp.float32)

        def body(i, carry):
            off = base + i * _C
            pltpu.sync_copy(src_hbm.at[pl.ds(off, _C)], src_v)
            pltpu.sync_copy(dst_hbm.at[pl.ds(off, _C)], dst_v)
            pltpu.async_copy(x_hbm.at[src_v], rows_v, sem).wait()
            pltpu.sync_copy(rows_v, acc_sh.at[dst_v], add=True)
            for k in range(_C // _L):
                idx16 = dst_v[pl.ds(k * _L, _L)]
                plsc.addupdate_scatter(cnt_v, [idx16], ones16)
            return carry

        lax.fori_loop(0, _CHUNKS, body, 0)

        pltpu.sync_copy(cnt_v, cnt_sh.at[sid])
        plsc.subcore_barrier()

        @pl.when(sid < _NWRITERS)
        def _():
            pltpu.sync_copy(acc_sh.at[pl.ds(sid * _RPT, _RPT)],
                            out_rows.at[cid, pl.ds(sid * _RPT, _RPT)])

        for r in (8, 4, 2, 1):
            @pl.when(sid < r)
            def _(r=r):
                pltpu.sync_copy(cnt_sh.at[sid + r], tmp_v)

                def rbody(k, carry):
                    sl = pl.ds(k * _L, _L)
                    cnt_v[sl] = cnt_v[sl] + tmp_v[sl]
                    return carry

                lax.fori_loop(0, _NVEC, rbody, 0)
                pltpu.sync_copy(cnt_v, cnt_sh.at[sid])
            plsc.subcore_barrier()

        @pl.when(sid == 0)
        def _():
            pltpu.sync_copy(cnt_v, out_cnt.at[cid])

    return seg_kernel(x, src, dst, zeros_blk)


_BN = 1000


def _tc_gates(acc, cnt, x, Wls, Wrs, bs, Wpi, Wpc, Wpo, bpi, bpc, bpo):
    def body(a, cn, xr, wls, wrs, bsr, wpi, wpc, wpo, bpir, bpcr, bpor,
             h_ref, c_ref):
        ssum = a[0] + a[1]
        n = jnp.maximum(cn[0] + cn[1], 1.0)
        mean = ssum / n
        u = (jnp.dot(mean, wls[:, :], preferred_element_type=jnp.float32)
             + jnp.dot(xr[:, :], wrs[:, :], preferred_element_type=jnp.float32)
             + bsr[:, :])
        zi = jnp.dot(u[:, 0:_H], wpi[:, :],
                     preferred_element_type=jnp.float32) + bpir[:, :]
        zc = jnp.dot(u[:, _H:2 * _H], wpc[:, :],
                     preferred_element_type=jnp.float32) + bpcr[:, :]
        zo = jnp.dot(u[:, 2 * _H:3 * _H], wpo[:, :],
                     preferred_element_type=jnp.float32) + bpor[:, :]
        ig = jax.nn.sigmoid(zi)
        tg = jnp.tanh(zc)
        og = jax.nn.sigmoid(zo)
        cnew = ig * tg
        c_ref[:, :] = cnew
        h_ref[:, :] = og * jnp.tanh(cnew)

    row_block = lambda w: pl.BlockSpec((_BN, w), lambda i: (i, 0))
    full = lambda arr: pl.BlockSpec(arr.shape, lambda i: (0,) * arr.ndim)

    return pl.pallas_call(
        body,
        grid=(_N // _BN,),
        in_specs=[
            pl.BlockSpec((_NC, _BN, _D), lambda i: (0, i, 0)),
            pl.BlockSpec((_NC, _BN, 1), lambda i: (0, i, 0)),
            row_block(_D),
            full(Wls), full(Wrs), full(bs),
            full(Wpi), full(Wpc), full(Wpo),
            full(bpi), full(bpc), full(bpo),
        ],
        out_specs=[row_block(_H), row_block(_H)],
        out_shape=[
            jax.ShapeDtypeStruct((_N, _H), jnp.float32),
            jax.ShapeDtypeStruct((_N, _H), jnp.float32),
        ],
    )(acc, cnt, x, Wls, Wrs, bs, Wpi, Wpc, Wpo, bpi, bpc, bpo)


def kernel(x, edge_index,
           Wl_i, Wr_i, b_i, Wp_i, bp_i,
           Wl_f, Wr_f, b_f, Wp_f, bp_f,
           Wl_c, Wr_c, b_c, Wp_c, bp_c,
           Wl_o, Wr_o, b_o, Wp_o, bp_o):
    src = edge_index[0].astype(jnp.int32)
    dst = edge_index[1].astype(jnp.int32)
    zeros_blk = jnp.zeros((_RPT, _D), jnp.float32)

    acc, cnt = _sc_segment_sum(x, src, dst, zeros_blk)
    cnt = cnt.reshape(_NC, _N, 1)

    Wls = jnp.concatenate([Wl_i, Wl_c, Wl_o], axis=1)
    Wrs = jnp.concatenate([Wr_i, Wr_c, Wr_o], axis=1)
    bs = jnp.concatenate([b_i, b_c, b_o])[None, :]

    h_new, c_new = _tc_gates(acc, cnt, x, Wls, Wrs, bs,
                             Wp_i, Wp_c, Wp_o,
                             bp_i[None, :], bp_c[None, :], bp_o[None, :])
    return (h_new, c_new)
